# baseline (speedup 1.0000x reference)
import functools

import jax
import jax.numpy as jnp
from jax import lax
from jax.experimental import pallas as pl
from jax.experimental.pallas import tpu as pltpu
from jax.experimental.pallas import tpu_sc as plsc

ROWS = 1024


def _dist_argmin_body(x2_ref, x_ref, wt_ref, w2_ref, idx_ref):
    x = x_ref[...]
    wt = wt_ref[...]
    p = lax.dot_general(x, wt, (((1,), (0,)), ((), ())),
                        preferred_element_type=jnp.float32)
    d2 = (x2_ref[...] + w2_ref[...]) - 2.0 * p
    dist = jnp.sqrt(jnp.maximum(d2, 0.0))
    k = dist.shape[1]
    m = jnp.min(dist, axis=1, keepdims=True)
    ks = lax.broadcasted_iota(jnp.int32, dist.shape, 1)
    idx_ref[...] = jnp.min(jnp.where(dist == m, ks, k), axis=1)


def _argmin_indices(xf, w2_row, wt, x2):
    n, d = xf.shape
    k = wt.shape[1]
    grid = n // ROWS
    return pl.pallas_call(
        _dist_argmin_body,
        grid=(grid,),
        in_specs=[
            pl.BlockSpec((ROWS, 1), lambda i: (i, 0)),
            pl.BlockSpec((ROWS, d), lambda i: (i, 0)),
            pl.BlockSpec((d, k), lambda i: (0, 0)),
            pl.BlockSpec((1, k), lambda i: (0, 0)),
        ],
        out_specs=pl.BlockSpec((ROWS,), lambda i: (i,)),
        out_shape=jax.ShapeDtypeStruct((n,), jnp.int32),
    )(x2, xf, wt, w2_row)


def _sc_gather(W, idx):
    n = idx.shape[0]
    d = W.shape[1]
    info = plsc.get_sparse_core_info()
    nw = info.num_cores * info.num_subcores
    b_per_w = n // nw
    chunk = 96
    nchunk = b_per_w // chunk
    mesh = plsc.VectorSubcoreMesh(core_axis_name="c", subcore_axis_name="s")

    @functools.partial(
        pl.kernel, mesh=mesh,
        compiler_params=pltpu.CompilerParams(use_tc_tiling_on_sc=False),
        out_type=jax.ShapeDtypeStruct((n, d), jnp.float32),
        scratch_types=[
            pltpu.VMEM((nchunk, chunk), jnp.int32),
            pltpu.VMEM((b_per_w, d), jnp.float32),
            pltpu.SemaphoreType.DMA,
        ],
    )
    def gather_kernel(table_hbm, idx_hbm, out_hbm, idx_v, rows_v, sem):
        wid = lax.axis_index("s") * info.num_cores + lax.axis_index("c")
        base = wid * b_per_w
        for c in range(nchunk):
            pltpu.sync_copy(idx_hbm.at[pl.ds(base + c * chunk, chunk)],
                            idx_v.at[c])
        copies = [
            pltpu.async_copy(table_hbm.at[idx_v.at[c]],
                             rows_v.at[pl.ds(c * chunk, chunk)], sem)
            for c in range(nchunk)
        ]
        for cp in copies:
            cp.wait()
        pltpu.sync_copy(rows_v, out_hbm.at[pl.ds(base, b_per_w)])

    return gather_kernel(W, idx)


def kernel(x, W):
    b, n, d = x.shape
    xf = x.reshape(b * n, d)
    x2 = jnp.sum(x * x, axis=-1, keepdims=True).reshape(b * n, 1)
    w2_row = jnp.sum(W * W, axis=-1)[None, :]
    wt = W.T
    idx = _argmin_indices(xf, w2_row, wt, x2)
    quantized = _sc_gather(W, idx)
    return quantized.reshape(b, n, d), idx.reshape(b, n)


# speedup vs baseline: 1.5065x; 1.5065x over previous
import jax
import jax.numpy as jnp
from jax import lax
from jax.experimental import pallas as pl

ROWS = 1024


def _vq_body(x2_ref, x_ref, wt2_ref, w2_ref, w_ref, idx_ref, q_ref):
    x = x_ref[...]
    wt2 = wt2_ref[...]
    p2 = lax.dot_general(x, wt2, (((1,), (0,)), ((), ())),
                         preferred_element_type=jnp.float32)
    d2 = (x2_ref[...] + w2_ref[...]) - p2
    dist = jnp.sqrt(jnp.maximum(d2, 0.0))
    k = dist.shape[1]
    m = jnp.min(dist, axis=1, keepdims=True)
    ksf = lax.broadcasted_iota(jnp.int32, dist.shape, 1).astype(jnp.float32)
    idxf = jnp.min(jnp.where(dist == m, ksf, float(k)), axis=1)
    idx_ref[...] = idxf.astype(jnp.int32)
    onehot = (ksf == idxf[:, None]).astype(jnp.float32)
    q_ref[...] = lax.dot_general(onehot, w_ref[...], (((1,), (0,)), ((), ())),
                                 preferred_element_type=jnp.float32)


def kernel(x, W):
    b, n, d = x.shape
    nk = W.shape[0]
    xf = x.reshape(b * n, d)
    x2 = jnp.sum(x * x, axis=-1, keepdims=True).reshape(b * n, 1)
    w2_row = jnp.sum(W * W, axis=-1)[None, :]
    wt2 = (2.0 * W).T
    grid = (b * n) // ROWS
    idx, quant = pl.pallas_call(
        _vq_body,
        grid=(grid,),
        in_specs=[
            pl.BlockSpec((ROWS, 1), lambda i: (i, 0)),
            pl.BlockSpec((ROWS, d), lambda i: (i, 0)),
            pl.BlockSpec((d, nk), lambda i: (0, 0)),
            pl.BlockSpec((1, nk), lambda i: (0, 0)),
            pl.BlockSpec((nk, d), lambda i: (0, 0)),
        ],
        out_specs=[
            pl.BlockSpec((ROWS,), lambda i: (i,)),
            pl.BlockSpec((ROWS, d), lambda i: (i, 0)),
        ],
        out_shape=[
            jax.ShapeDtypeStruct((b * n,), jnp.int32),
            jax.ShapeDtypeStruct((b * n, d), jnp.float32),
        ],
    )(x2, xf, wt2, w2_row, W)
    return quant.reshape(b, n, d), idx.reshape(b, n)
